# baseline (speedup 1.0000x reference)
import jax
import jax.numpy as jnp
import numpy as np
from jax.experimental import pallas as pl
from jax.experimental.pallas import tpu as pltpu

_NE = 5
_TK = 3
_NC = 10
_B = 128

_P1 = [(j, g, 2 * (g % 16) + g // 16 + j)
       for j in range(3) for g in range(32)
       if g % 16 <= 12 and 2 * (g % 16) + g // 16 <= 25]

_P2A = [(j, (2 * xx2 + a + j) * 32, a * 128 + xx2 * 64)
        for j in range(3) for a in range(2) for xx2 in range(2)]
_P2B = [(j, (2 * xx2 + a + j - 4) * 32, a * 256 + (xx2 - 2) * 64)
        for j in range(3) for a in range(2) for xx2 in range(2, 5)]

_OH31 = np.zeros((1, 32), np.float32)
_OH31[0, 31] = 1.0


def _body(x_ref, w1_ref, b1_ref, w2_ref, b2_ref, wt_ref, gb_ref,
          eb_ref, out_ref, w1s, w2sa, w2sb):
    f32 = jnp.float32
    bf16 = jnp.bfloat16

    w1s[...] = jnp.zeros((128, 1024), bf16)
    w1v = {(i, j): w1_ref[i, j, 0:1, :].astype(bf16)
           for i in range(3) for j in range(3)}
    for i in range(3):
        for (j, g, xx) in _P1:
            w1s[pl.ds(i * 32 + xx, 1), pl.ds(g * 32, 32)] = w1v[(i, j)]
    b1v = b1_ref[...][None, :].astype(bf16)
    for g in range(32):
        if g % 16 <= 12:
            w1s[pl.ds(96, 1), pl.ds(g * 32, 32)] = b1v

    w2sa[...] = jnp.zeros((768, 256), bf16)
    w2sb[...] = jnp.zeros((768, 512), bf16)
    w2v = {(i, j): w2_ref[i, j, :, :].astype(bf16)
           for i in range(3) for j in range(3)}
    for i in range(3):
        for (j, rb, cb) in _P2A:
            w2sa[pl.ds(i * 256 + rb, 32), pl.ds(cb, 64)] = w2v[(i, j)]
        for (j, rb, cb) in _P2B:
            w2sb[pl.ds(i * 256 + rb, 32), pl.ds(cb, 64)] = w2v[(i, j)]

    xc = x_ref[...]

    c1 = jnp.dot(xc, w1s[...],
                 preferred_element_type=f32).astype(bf16)
    p = jnp.maximum(jnp.maximum(c1[:, :512], c1[:, 512:]),
                    jnp.bfloat16(0.0))

    p3 = p.reshape(13, 2, 128, 512)
    q2 = jnp.maximum(p3[:, 0], p3[:, 1]).reshape(1664, 512)

    ga = jnp.concatenate(
        [q2[0:1408, 0:256], q2[128:1536, 0:256], q2[256:1664, 0:256]],
        axis=1)
    gb = jnp.concatenate(
        [q2[0:1408, 128:384], q2[128:1536, 128:384], q2[256:1664, 128:384]],
        axis=1)
    o2a = jnp.dot(ga, w2sa[...], preferred_element_type=f32)
    o2b = jnp.dot(gb, w2sb[...], preferred_element_type=f32)
    pa = jnp.maximum(o2a[:, 0:128], o2a[:, 128:256])
    pb = jnp.maximum(o2b[:, 0:256], o2b[:, 256:512])
    b2v = b2_ref[...][None, :]
    b2t = jnp.concatenate([b2v] * 6, axis=1)
    p2 = jnp.maximum(jnp.concatenate([pa, pb], axis=1) + b2t, 0.0)

    p2r = p2[0:1280].reshape(5, 2, 128, 384)
    h2 = jnp.maximum(p2r[:, 0], p2r[:, 1]).astype(bf16)

    wt = wt_ref[...]
    out55 = jnp.dot(h2[0][:, 0:320], jnp.transpose(wt[:, 0:320], (1, 0)),
                    preferred_element_type=f32)
    for k in range(1, 5):
        out55 = out55 + jnp.dot(
            h2[k][:, 0:320],
            jnp.transpose(wt[:, 320 * k:320 * (k + 1)], (1, 0)),
            preferred_element_type=f32)
    g5 = out55[:, 0:5] + gb_ref[...][None, :]

    g5 = g5 - jnp.max(g5, axis=1, keepdims=True)
    eg = jnp.exp(g5)
    gate = eg / jnp.sum(eg, axis=1, keepdims=True)

    ebv = eb_ref[...]
    acc = jnp.zeros((128, _NC), dtype=f32)
    for e in range(_NE):
        ge = gate[:, e:e + 1]
        better = (gate > ge).astype(f32)
        if e > 0:
            tie_lt = (jnp.arange(5) < e).astype(f32)
            better = better + (gate == ge).astype(f32) * tie_lt[None, :]
        rank = jnp.sum(better, axis=1, keepdims=True)
        keep = (rank < float(_TK)).astype(f32)
        eo = out55[:, 5 + _NC * e: 5 + _NC * (e + 1)] + ebv[e:e + 1, :]
        acc = acc + keep * ge * eo

    acc = acc - jnp.max(acc, axis=1, keepdims=True)
    ea = jnp.exp(acc)
    out_ref[...] = ea / jnp.sum(ea, axis=1, keepdims=True)


def kernel(inputs, conv1_w, conv1_b, conv2_w, conv2_b, gate_w, gate_b,
           expert_w, expert_b):
    bf16 = jnp.bfloat16

    xp = jnp.pad(inputs[..., 0], ((0, 0), (0, 0), (0, 4)))
    xt = jnp.transpose(xp, (1, 0, 2))
    ones = jnp.ones((26, 128, 32), inputs.dtype)
    xcat = jnp.concatenate([xt[0:26], xt[1:27], xt[2:28], ones],
                           axis=2).reshape(3328, 128).astype(bf16)

    wallt = jnp.concatenate(
        [jnp.transpose(gate_w, (1, 0)),
         jnp.transpose(expert_w, (0, 2, 1)).reshape(50, 1600)],
        axis=0).astype(bf16)

    return pl.pallas_call(
        _body,
        out_shape=jax.ShapeDtypeStruct((_B, _NC), jnp.float32),
        scratch_shapes=[
            pltpu.VMEM((128, 1024), jnp.bfloat16),
            pltpu.VMEM((768, 256), jnp.bfloat16),
            pltpu.VMEM((768, 512), jnp.bfloat16),
        ],
    )(xcat, conv1_w, conv1_b, conv2_w, conv2_b, wallt, gate_b, expert_b)


# speedup vs baseline: 1.0002x; 1.0002x over previous
import jax
import jax.numpy as jnp
import numpy as np
from jax.experimental import pallas as pl
from jax.experimental.pallas import tpu as pltpu

_NE = 5
_TK = 3
_NC = 10
_B = 128

_P1 = [(j, g, 2 * (g % 16) + g // 16 + j)
       for j in range(3) for g in range(32)
       if g % 16 <= 12 and 2 * (g % 16) + g // 16 <= 25]

_P2A = [(j, (2 * xx2 + a + j) * 32, a * 128 + xx2 * 64)
        for j in range(3) for a in range(2) for xx2 in range(2)]
_P2B = [(j, (2 * xx2 + a + j - 4) * 32, a * 256 + (xx2 - 2) * 64)
        for j in range(3) for a in range(2) for xx2 in range(2, 5)]

def _body(x_ref, w1_ref, b1_ref, w2_ref, b2_ref, wt_ref, gb_ref,
          eb_ref, out_ref, w1s, w2sa, w2sb):
    f32 = jnp.float32
    bf16 = jnp.bfloat16

    w1s[...] = jnp.zeros((128, 1024), bf16)
    w1v = {(i, j): w1_ref[i, j, 0:1, :].astype(bf16)
           for i in range(3) for j in range(3)}
    for i in range(3):
        for (j, g, xx) in _P1:
            w1s[pl.ds(i * 32 + xx, 1), pl.ds(g * 32, 32)] = w1v[(i, j)]
    b1v = b1_ref[...][None, :].astype(bf16)
    for g in range(32):
        if g % 16 <= 12:
            w1s[pl.ds(96, 1), pl.ds(g * 32, 32)] = b1v

    w2sa[...] = jnp.zeros((768, 256), bf16)
    w2sb[...] = jnp.zeros((768, 512), bf16)
    w2v = {(i, j): w2_ref[i, j, :, :].astype(bf16)
           for i in range(3) for j in range(3)}
    for i in range(3):
        for (j, rb, cb) in _P2A:
            w2sa[pl.ds(i * 256 + rb, 32), pl.ds(cb, 64)] = w2v[(i, j)]
        for (j, rb, cb) in _P2B:
            w2sb[pl.ds(i * 256 + rb, 32), pl.ds(cb, 64)] = w2v[(i, j)]

    xc = x_ref[...]

    c1 = jnp.dot(xc, w1s[...],
                 preferred_element_type=f32).astype(bf16)
    p = jnp.maximum(jnp.maximum(c1[:, :512], c1[:, 512:]),
                    jnp.bfloat16(0.0))

    p3 = p.reshape(13, 2, 128, 512)
    q2 = jnp.maximum(p3[:, 0], p3[:, 1]).reshape(1664, 512)

    ga = jnp.concatenate(
        [q2[0:1408, 0:256], q2[128:1536, 0:256], q2[256:1664, 0:256]],
        axis=1)
    gb = jnp.concatenate(
        [q2[0:1408, 128:384], q2[128:1536, 128:384], q2[256:1664, 128:384]],
        axis=1)
    o2a = jnp.dot(ga, w2sa[...], preferred_element_type=f32)
    o2b = jnp.dot(gb, w2sb[...], preferred_element_type=f32)
    pa = jnp.maximum(o2a[:, 0:128], o2a[:, 128:256])
    pb = jnp.maximum(o2b[:, 0:256], o2b[:, 256:512])
    b2v = b2_ref[...][None, :]
    b2t = jnp.concatenate([b2v] * 6, axis=1)
    p2 = jnp.maximum(jnp.concatenate([pa, pb], axis=1) + b2t, 0.0)

    p2r = p2[0:1280].reshape(5, 2, 128, 384)
    h2 = jnp.maximum(p2r[:, 0], p2r[:, 1]).astype(bf16)

    wt = wt_ref[...]
    out55 = jnp.dot(h2[0][:, 0:320], jnp.transpose(wt[:, 0:320], (1, 0)),
                    preferred_element_type=f32)
    for k in range(1, 5):
        out55 = out55 + jnp.dot(
            h2[k][:, 0:320],
            jnp.transpose(wt[:, 320 * k:320 * (k + 1)], (1, 0)),
            preferred_element_type=f32)
    g5 = out55[:, 0:5] + gb_ref[...][None, :]

    g5 = g5 - jnp.max(g5, axis=1, keepdims=True)
    eg = jnp.exp(g5)
    gate = eg / jnp.sum(eg, axis=1, keepdims=True)

    ebv = eb_ref[...]
    acc = jnp.zeros((128, _NC), dtype=f32)
    for e in range(_NE):
        ge = gate[:, e:e + 1]
        better = (gate > ge).astype(f32)
        if e > 0:
            tie_lt = (jnp.arange(5) < e).astype(f32)
            better = better + (gate == ge).astype(f32) * tie_lt[None, :]
        rank = jnp.sum(better, axis=1, keepdims=True)
        keep = (rank < float(_TK)).astype(f32)
        eo = out55[:, 5 + _NC * e: 5 + _NC * (e + 1)] + ebv[e:e + 1, :]
        acc = acc + keep * ge * eo

    acc = acc - jnp.max(acc, axis=1, keepdims=True)
    ea = jnp.exp(acc)
    out_ref[...] = ea / jnp.sum(ea, axis=1, keepdims=True)


def kernel(inputs, conv1_w, conv1_b, conv2_w, conv2_b, gate_w, gate_b,
           expert_w, expert_b):
    bf16 = jnp.bfloat16

    xp = jnp.pad(inputs[..., 0], ((0, 0), (0, 0), (0, 4)))
    xt = jnp.transpose(xp, (1, 0, 2))
    ones = jnp.ones((26, 128, 32), inputs.dtype)
    xcat = jnp.concatenate([xt[0:26], xt[1:27], xt[2:28], ones],
                           axis=2).reshape(3328, 128).astype(bf16)

    wallt = jnp.concatenate(
        [jnp.transpose(gate_w, (1, 0)),
         jnp.transpose(expert_w, (0, 2, 1)).reshape(50, 1600)],
        axis=0).astype(bf16)

    return pl.pallas_call(
        _body,
        out_shape=jax.ShapeDtypeStruct((_B, _NC), jnp.float32),
        scratch_shapes=[
            pltpu.VMEM((128, 1024), jnp.bfloat16),
            pltpu.VMEM((768, 256), jnp.bfloat16),
            pltpu.VMEM((768, 512), jnp.bfloat16),
        ],
    )(xcat, conv1_w, conv1_b, conv2_w, conv2_b, wallt, gate_b, expert_b)
